# baseline (speedup 1.0000x reference)
import functools

import jax
import jax.numpy as jnp
from jax import lax
from jax.experimental import pallas as pl
from jax.experimental.pallas import tpu as pltpu
from jax.experimental.pallas import tpu_sc as plsc

_N = 200
_F = 512
_H = 256
_G = 4 * _H
_TSEQ = 16
_T = 192
_K = 30
_ROWW = 224
_CW = _N * _ROWW
_NWIN = 10
_WIN = 128
_EPT = _NWIN * _WIN
_NTILES = 32
_EPAD = _EPT * _NTILES
_ZCH = _CW // 16
_PHI = jax.lax.Precision.HIGHEST


def _sc_counts_body(idx_hbm, out_hbm, idx_v, upd_v, zeros_v, csp):
    c = lax.axis_index("c")
    s = lax.axis_index("s")

    ones16 = jnp.ones((16,), jnp.float32)
    for j in range(_NWIN):
        for i in range(_WIN // 16):
            upd_v[j, pl.ds(i * 16, 16)] = ones16

    z16 = jnp.zeros((16,), jnp.float32)
    for i in range(_ZCH // 16):
        zeros_v[pl.ds(i * 16, 16)] = z16
    pltpu.sync_copy(zeros_v, csp.at[pl.ds(s * _ZCH, _ZCH)])

    plsc.subcore_barrier()

    wid = c * 16 + s
    pltpu.sync_copy(idx_hbm.at[wid], idx_v)
    for j in range(_NWIN):
        pltpu.sync_copy(upd_v.at[j], csp.at[idx_v.at[j]], add=True)

    plsc.subcore_barrier()

    @pl.when(s == 0)
    def _flush():
        pltpu.sync_copy(csp, out_hbm.at[c])


@functools.cache
def _get_sc_counts():
    return pl.kernel(
        _sc_counts_body,
        out_type=jax.ShapeDtypeStruct((2, _CW), jnp.float32),
        mesh=plsc.VectorSubcoreMesh(core_axis_name="c", subcore_axis_name="s"),
        scratch_types=[
            pltpu.VMEM((_NWIN, _WIN), jnp.int32),
            pltpu.VMEM((_NWIN, _WIN), jnp.float32),
            pltpu.VMEM((_ZCH,), jnp.float32),
            pltpu.VMEM_SHARED((_CW,), jnp.float32),
        ],
    )


def _main_body(cp_ref, x_ref,
               wxi_ref, wxf_ref, wxo_ref, wxm_ref,
               bxi_ref, bxf_ref, bxo_ref, bxm_ref,
               whi_ref, whf_ref, who_ref, whm_ref,
               bhi_ref, bhf_ref, bho_ref, bhm_ref,
               h0_ref, c0_ref, pv_ref, ts_ref,
               wih_ref, whh_ref, lbih_ref, lbhh_ref,
               lng_ref, lnb_ref, m1w_ref, m1b_ref, m2w_ref, m2b_ref,
               pred_ref, ploss_ref):
    f32 = jnp.float32
    nt = (((1,), (1,)), ((), ()))

    cm = cp_ref[0] + cp_ref[1]
    ct = cm[:, :_N]
    deg = 1.0 + jnp.sum(ct, axis=1, keepdims=True)
    dinv = lax.rsqrt(deg)
    rr = lax.broadcasted_iota(jnp.int32, (_N, _N), 0)
    cc = lax.broadcasted_iota(jnp.int32, (_N, _N), 1)
    eye = (rr == cc).astype(f32)
    dinv_row = lax.dot_general(dinv, eye, (((0,), (0,)), ((), ())),
                               preferred_element_type=f32, precision=_PHI)
    adj = ct * dinv * dinv_row + eye * (dinv * dinv)

    wx = jnp.concatenate(
        [wxi_ref[...], wxf_ref[...], wxo_ref[...], wxm_ref[...]], axis=1)
    wh = jnp.concatenate(
        [whi_ref[...], whf_ref[...], who_ref[...], whm_ref[...]], axis=1)
    bg = jnp.concatenate(
        [bxi_ref[...] + bhi_ref[...], bxf_ref[...] + bhf_ref[...],
         bxo_ref[...] + bho_ref[...], bxm_ref[...] + bhm_ref[...]], axis=1)

    xws = []
    for t in range(_TSEQ):
        x = x_ref[t]
        mu = jnp.mean(x, axis=0, keepdims=True)
        xc = x - mu
        sd = jnp.sqrt(jnp.sum(xc * xc, axis=0, keepdims=True) * (1.0 / (_N - 1)))
        xn = xc / (sd + 1e-6)
        xws.append(jnp.dot(xn, wx, preferred_element_type=f32, precision=_PHI))

    g0 = (lax.dot_general(ts_ref[...], wih_ref[...], nt,
                          preferred_element_type=f32, precision=_PHI)
          + lbih_ref[...] + lbhh_ref[...])
    whh = whh_ref[...]

    lstm = {"h": jnp.zeros((1, _H), f32), "c": jnp.zeros((1, _H), f32)}

    def lstm_step(t):
        g = g0[t:t + 1, :] + lax.dot_general(lstm["h"], whh, nt,
                                             preferred_element_type=f32, precision=_PHI)
        i = jax.nn.sigmoid(g[:, :_H])
        f = jax.nn.sigmoid(g[:, _H:2 * _H])
        gg = jnp.tanh(g[:, 2 * _H:3 * _H])
        o = jax.nn.sigmoid(g[:, 3 * _H:])
        lstm["c"] = f * lstm["c"] + i * gg
        lstm["h"] = o * jnp.tanh(lstm["c"])

    h = h0_ref[...]
    c = c0_ref[...]
    lsteps_per = _T // _TSEQ
    for t in range(_TSEQ):
        p = xws[t] + jnp.dot(h, wh, preferred_element_type=f32, precision=_PHI)
        z = jnp.dot(adj, p, preferred_element_type=f32, precision=_PHI) + bg
        ig = jax.nn.sigmoid(z[:, :_H])
        fg = jax.nn.sigmoid(z[:, _H:2 * _H])
        og = jax.nn.sigmoid(z[:, 2 * _H:3 * _H])
        mg = jnp.maximum(z[:, 3 * _H:], 0.0)
        c = jnp.tanh(ig * mg + fg * c)
        h = og * jnp.tanh(c)
        for k in range(lsteps_per):
            lstm_step(t * lsteps_per + k)

    pv = pv_ref[...]
    n2 = jnp.sqrt(jnp.sum(pv * pv))
    scol = (jnp.dot(h, pv, preferred_element_type=f32, precision=_PHI)
            * (1.0 / (n2 + 1e-8)))
    mu_s = jnp.mean(scol)
    sd_s = jnp.sqrt(jnp.mean((scol - mu_s) ** 2))
    sig = jax.nn.sigmoid((scol - mu_s) / (sd_s + 1e-8))
    ploss_ref[...] = jnp.mean(sig * (1.0 - sig)).reshape(1, 1)

    srow = lax.dot_general(sig, eye, (((0,), (0,)), ((), ())),
                           preferred_element_type=f32, precision=_PHI)
    gt = (srow > sig).astype(f32)
    eq = (srow == sig).astype(f32)
    lower = (cc < rr).astype(f32)
    rank = jnp.sum(gt + eq * lower, axis=1, keepdims=True)
    selc = (rank < float(_K)).astype(f32)
    selrow = lax.dot_general(selc, eye, (((0,), (0,)), ((), ())),
                             preferred_element_type=f32, precision=_PHI)
    xs = h * sig
    high = jnp.dot(selrow, xs, preferred_element_type=f32, precision=_PHI) * (1.0 / _K)

    fusion = jnp.concatenate([high, lstm["h"]], axis=1)
    mu = jnp.mean(fusion)
    var = jnp.mean((fusion - mu) ** 2)
    fn = (fusion - mu) / jnp.sqrt(var + 1e-5) * lng_ref[...] + lnb_ref[...]
    hmid = jnp.maximum(
        jnp.dot(fn, m1w_ref[...], preferred_element_type=f32, precision=_PHI) + m1b_ref[...],
        0.0)
    pred_ref[...] = (jnp.dot(hmid, m2w_ref[...], preferred_element_type=f32, precision=_PHI)
                     + m2b_ref[...])


_main_call = pl.pallas_call(
    _main_body,
    out_shape=[
        jax.ShapeDtypeStruct((1, 1), jnp.float32),
        jax.ShapeDtypeStruct((1, 1), jnp.float32),
    ],
)


@jax.jit
def kernel(lw_matrixes_sequence, edge_index, hidden_state, cell_state,
           time_series, Wi_x, Wf_x, Wo_x, Wm_x, bi_x, bf_x, bo_x, bm_x,
           Wi_h, Wf_h, Wo_h, Wm_h, bi_h, bf_h, bo_h, bm_h,
           ln_g, ln_b, pool_v, lstm_Wih, lstm_Whh, lstm_bih, lstm_bhh,
           mlp1_W, mlp1_b, mlp2_W, mlp2_b):
    e = edge_index.shape[1]
    eidx = edge_index[1] * _ROWW + edge_index[0]
    npad = _EPAD - e
    j = jnp.arange(npad, dtype=jnp.int32)
    pad_idx = (j % _N) * _ROWW + _N + (j // _N)
    idx_full = jnp.concatenate([eidx, pad_idx]).reshape(_NTILES, _NWIN, _WIN)

    cp = _get_sc_counts()(idx_full).reshape(2, _N, _ROWW)

    r1 = lambda b: b.reshape(1, b.shape[0])
    pred, ploss = _main_call(
        cp, lw_matrixes_sequence,
        Wi_x, Wf_x, Wo_x, Wm_x,
        r1(bi_x), r1(bf_x), r1(bo_x), r1(bm_x),
        Wi_h, Wf_h, Wo_h, Wm_h,
        r1(bi_h), r1(bf_h), r1(bo_h), r1(bm_h),
        hidden_state, cell_state, pool_v, time_series,
        lstm_Wih, lstm_Whh, r1(lstm_bih), r1(lstm_bhh),
        r1(ln_g), r1(ln_b), mlp1_W, r1(mlp1_b), mlp2_W, r1(mlp2_b))

    return pred.reshape(1), ploss[0, 0]


# speedup vs baseline: 3.0321x; 3.0321x over previous
import functools

import jax
import jax.numpy as jnp
from jax import lax
from jax.experimental import pallas as pl
from jax.experimental.pallas import tpu as pltpu
from jax.experimental.pallas import tpu_sc as plsc

_N = 200
_F = 512
_H = 256
_G = 4 * _H
_TSEQ = 16
_T = 192
_K = 30
_ROWW = 224
_CW = _N * _ROWW
_NWIN = 10
_WIN = 128
_EPT = _NWIN * _WIN
_NTILES = 32
_EPAD = _EPT * _NTILES
_ZCH = _CW // 16
_PHI = jax.lax.Precision.DEFAULT


def _sc_counts_body(idx_hbm, out_hbm, idx_v, upd_v, zeros_v, csp):
    c = lax.axis_index("c")
    s = lax.axis_index("s")

    ones16 = jnp.ones((16,), jnp.float32)
    for j in range(_NWIN):
        for i in range(_WIN // 16):
            upd_v[j, pl.ds(i * 16, 16)] = ones16

    z16 = jnp.zeros((16,), jnp.float32)
    for i in range(_ZCH // 16):
        zeros_v[pl.ds(i * 16, 16)] = z16
    pltpu.sync_copy(zeros_v, csp.at[pl.ds(s * _ZCH, _ZCH)])

    plsc.subcore_barrier()

    wid = c * 16 + s
    pltpu.sync_copy(idx_hbm.at[wid], idx_v)
    for j in range(_NWIN):
        pltpu.sync_copy(upd_v.at[j], csp.at[idx_v.at[j]], add=True)

    plsc.subcore_barrier()

    @pl.when(s == 0)
    def _flush():
        pltpu.sync_copy(csp, out_hbm.at[c])


@functools.cache
def _get_sc_counts():
    return pl.kernel(
        _sc_counts_body,
        out_type=jax.ShapeDtypeStruct((2, _CW), jnp.float32),
        mesh=plsc.VectorSubcoreMesh(core_axis_name="c", subcore_axis_name="s"),
        scratch_types=[
            pltpu.VMEM((_NWIN, _WIN), jnp.int32),
            pltpu.VMEM((_NWIN, _WIN), jnp.float32),
            pltpu.VMEM((_ZCH,), jnp.float32),
            pltpu.VMEM_SHARED((_CW,), jnp.float32),
        ],
    )


def _main_body(cp_ref, x_ref,
               wxi_ref, wxf_ref, wxo_ref, wxm_ref,
               bxi_ref, bxf_ref, bxo_ref, bxm_ref,
               whi_ref, whf_ref, who_ref, whm_ref,
               bhi_ref, bhf_ref, bho_ref, bhm_ref,
               h0_ref, c0_ref, pv_ref, ts_ref,
               wih_ref, whh_ref, lbih_ref, lbhh_ref,
               lng_ref, lnb_ref, m1w_ref, m1b_ref, m2w_ref, m2b_ref,
               pred_ref, ploss_ref):
    f32 = jnp.float32
    nt = (((1,), (1,)), ((), ()))

    cm = cp_ref[0] + cp_ref[1]
    ct = cm[:, :_N]
    deg = 1.0 + jnp.sum(ct, axis=1, keepdims=True)
    dinv = lax.rsqrt(deg)
    rr = lax.broadcasted_iota(jnp.int32, (_N, _N), 0)
    cc = lax.broadcasted_iota(jnp.int32, (_N, _N), 1)
    eye = (rr == cc).astype(f32)
    dinv_row = lax.dot_general(dinv, eye, (((0,), (0,)), ((), ())),
                               preferred_element_type=f32, precision=_PHI)
    adj = ct * dinv * dinv_row + eye * (dinv * dinv)

    wx = jnp.concatenate(
        [wxi_ref[...], wxf_ref[...], wxo_ref[...], wxm_ref[...]], axis=1)
    wh = jnp.concatenate(
        [whi_ref[...], whf_ref[...], who_ref[...], whm_ref[...]], axis=1)
    bg = jnp.concatenate(
        [bxi_ref[...] + bhi_ref[...], bxf_ref[...] + bhf_ref[...],
         bxo_ref[...] + bho_ref[...], bxm_ref[...] + bhm_ref[...]], axis=1)

    xws = []
    for t in range(_TSEQ):
        x = x_ref[t]
        mu = jnp.mean(x, axis=0, keepdims=True)
        xc = x - mu
        sd = jnp.sqrt(jnp.sum(xc * xc, axis=0, keepdims=True) * (1.0 / (_N - 1)))
        xn = xc / (sd + 1e-6)
        xws.append(jnp.dot(xn, wx, preferred_element_type=f32, precision=_PHI))

    g0 = (lax.dot_general(ts_ref[...], wih_ref[...], nt,
                          preferred_element_type=f32, precision=_PHI)
          + lbih_ref[...] + lbhh_ref[...])
    whh = whh_ref[...]

    lstm = {"h": jnp.zeros((1, _H), f32), "c": jnp.zeros((1, _H), f32)}

    def lstm_step(t):
        g = g0[t:t + 1, :] + lax.dot_general(lstm["h"], whh, nt,
                                             preferred_element_type=f32, precision=_PHI)
        i = jax.nn.sigmoid(g[:, :_H])
        f = jax.nn.sigmoid(g[:, _H:2 * _H])
        gg = jnp.tanh(g[:, 2 * _H:3 * _H])
        o = jax.nn.sigmoid(g[:, 3 * _H:])
        lstm["c"] = f * lstm["c"] + i * gg
        lstm["h"] = o * jnp.tanh(lstm["c"])

    h = h0_ref[...]
    c = c0_ref[...]
    lsteps_per = _T // _TSEQ
    for t in range(_TSEQ):
        p = xws[t] + jnp.dot(h, wh, preferred_element_type=f32, precision=_PHI)
        z = jnp.dot(adj, p, preferred_element_type=f32, precision=_PHI) + bg
        ig = jax.nn.sigmoid(z[:, :_H])
        fg = jax.nn.sigmoid(z[:, _H:2 * _H])
        og = jax.nn.sigmoid(z[:, 2 * _H:3 * _H])
        mg = jnp.maximum(z[:, 3 * _H:], 0.0)
        c = jnp.tanh(ig * mg + fg * c)
        h = og * jnp.tanh(c)
        for k in range(lsteps_per):
            lstm_step(t * lsteps_per + k)

    pv = pv_ref[...]
    n2 = jnp.sqrt(jnp.sum(pv * pv))
    scol = (jnp.dot(h, pv, preferred_element_type=f32, precision=_PHI)
            * (1.0 / (n2 + 1e-8)))
    mu_s = jnp.mean(scol)
    sd_s = jnp.sqrt(jnp.mean((scol - mu_s) ** 2))
    sig = jax.nn.sigmoid((scol - mu_s) / (sd_s + 1e-8))
    ploss_ref[...] = jnp.mean(sig * (1.0 - sig)).reshape(1, 1)

    srow = lax.dot_general(sig, eye, (((0,), (0,)), ((), ())),
                           preferred_element_type=f32, precision=_PHI)
    gt = (srow > sig).astype(f32)
    eq = (srow == sig).astype(f32)
    lower = (cc < rr).astype(f32)
    rank = jnp.sum(gt + eq * lower, axis=1, keepdims=True)
    selc = (rank < float(_K)).astype(f32)
    selrow = lax.dot_general(selc, eye, (((0,), (0,)), ((), ())),
                             preferred_element_type=f32, precision=_PHI)
    xs = h * sig
    high = jnp.dot(selrow, xs, preferred_element_type=f32, precision=_PHI) * (1.0 / _K)

    fusion = jnp.concatenate([high, lstm["h"]], axis=1)
    mu = jnp.mean(fusion)
    var = jnp.mean((fusion - mu) ** 2)
    fn = (fusion - mu) / jnp.sqrt(var + 1e-5) * lng_ref[...] + lnb_ref[...]
    hmid = jnp.maximum(
        jnp.dot(fn, m1w_ref[...], preferred_element_type=f32, precision=_PHI) + m1b_ref[...],
        0.0)
    pred_ref[...] = (jnp.dot(hmid, m2w_ref[...], preferred_element_type=f32, precision=_PHI)
                     + m2b_ref[...])


_main_call = pl.pallas_call(
    _main_body,
    out_shape=[
        jax.ShapeDtypeStruct((1, 1), jnp.float32),
        jax.ShapeDtypeStruct((1, 1), jnp.float32),
    ],
)


@jax.jit
def kernel(lw_matrixes_sequence, edge_index, hidden_state, cell_state,
           time_series, Wi_x, Wf_x, Wo_x, Wm_x, bi_x, bf_x, bo_x, bm_x,
           Wi_h, Wf_h, Wo_h, Wm_h, bi_h, bf_h, bo_h, bm_h,
           ln_g, ln_b, pool_v, lstm_Wih, lstm_Whh, lstm_bih, lstm_bhh,
           mlp1_W, mlp1_b, mlp2_W, mlp2_b):
    e = edge_index.shape[1]
    eidx = edge_index[1] * _ROWW + edge_index[0]
    npad = _EPAD - e
    j = jnp.arange(npad, dtype=jnp.int32)
    pad_idx = (j % _N) * _ROWW + _N + (j // _N)
    idx_full = jnp.concatenate([eidx, pad_idx]).reshape(_NTILES, _NWIN, _WIN)

    cp = _get_sc_counts()(idx_full).reshape(2, _N, _ROWW)

    r1 = lambda b: b.reshape(1, b.shape[0])
    pred, ploss = _main_call(
        cp, lw_matrixes_sequence,
        Wi_x, Wf_x, Wo_x, Wm_x,
        r1(bi_x), r1(bf_x), r1(bo_x), r1(bm_x),
        Wi_h, Wf_h, Wo_h, Wm_h,
        r1(bi_h), r1(bf_h), r1(bo_h), r1(bm_h),
        hidden_state, cell_state, pool_v, time_series,
        lstm_Wih, lstm_Whh, r1(lstm_bih), r1(lstm_bhh),
        r1(ln_g), r1(ln_b), mlp1_W, r1(mlp1_b), mlp2_W, r1(mlp2_b))

    return pred.reshape(1), ploss[0, 0]


# speedup vs baseline: 3.2065x; 1.0575x over previous
import functools

import jax
import jax.numpy as jnp
from jax import lax
from jax.experimental import pallas as pl
from jax.experimental.pallas import tpu as pltpu
from jax.experimental.pallas import tpu_sc as plsc

_N = 200
_F = 512
_H = 256
_G = 4 * _H
_TSEQ = 16
_T = 192
_K = 30
_ROWW = 224
_CW = _N * _ROWW
_NWIN = 10
_WIN = 128
_EPT = _NWIN * _WIN
_NTILES = 32
_EPAD = _EPT * _NTILES
_ZCH = _CW // 16
_PHI = jax.lax.Precision.DEFAULT


def _sc_counts_body(idx_hbm, out_hbm, idx_v, upd_v, zeros_v, csp):
    c = lax.axis_index("c")
    s = lax.axis_index("s")

    ones16 = jnp.ones((16,), jnp.float32)
    for j in range(_NWIN):
        for i in range(_WIN // 16):
            upd_v[j, pl.ds(i * 16, 16)] = ones16

    z16 = jnp.zeros((16,), jnp.float32)
    for i in range(_ZCH // 16):
        zeros_v[pl.ds(i * 16, 16)] = z16
    pltpu.sync_copy(zeros_v, csp.at[pl.ds(s * _ZCH, _ZCH)])

    plsc.subcore_barrier()

    wid = c * 16 + s
    pltpu.sync_copy(idx_hbm.at[wid], idx_v)
    for j in range(_NWIN):
        pltpu.sync_copy(upd_v.at[j], csp.at[idx_v.at[j]], add=True)

    plsc.subcore_barrier()

    @pl.when(s == 0)
    def _flush():
        pltpu.sync_copy(csp, out_hbm.at[c])


@functools.cache
def _get_sc_counts():
    return pl.kernel(
        _sc_counts_body,
        out_type=jax.ShapeDtypeStruct((2, _CW), jnp.float32),
        mesh=plsc.VectorSubcoreMesh(core_axis_name="c", subcore_axis_name="s"),
        scratch_types=[
            pltpu.VMEM((_NWIN, _WIN), jnp.int32),
            pltpu.VMEM((_NWIN, _WIN), jnp.float32),
            pltpu.VMEM((_ZCH,), jnp.float32),
            pltpu.VMEM_SHARED((_CW,), jnp.float32),
        ],
    )


def _main_body(cp_ref, x_ref, wx_ref, wh_ref, bg_ref,
               h0_ref, c0_ref, pv_ref, ts_ref,
               wiht_ref, whht_ref, lb_ref,
               lng_ref, lnb_ref, m1w_ref, m1b_ref, m2w_ref, m2b_ref,
               pred_ref, ploss_ref):
    f32 = jnp.float32

    cm = cp_ref[0] + cp_ref[1]
    ct = cm[:, :_N]
    deg = 1.0 + jnp.sum(ct, axis=1, keepdims=True)
    dinv = lax.rsqrt(deg)
    rr = lax.broadcasted_iota(jnp.int32, (_N, _N), 0)
    cc = lax.broadcasted_iota(jnp.int32, (_N, _N), 1)
    eye = (rr == cc).astype(f32)
    dinv_row = lax.dot_general(dinv, eye, (((0,), (0,)), ((), ())),
                               preferred_element_type=f32, precision=_PHI)
    adj = ct * dinv * dinv_row + eye * (dinv * dinv)

    wx = wx_ref[...]
    wh = wh_ref[...]
    bg = bg_ref[...]

    xws = []
    for t in range(_TSEQ):
        x = x_ref[t]
        mu = jnp.mean(x, axis=0, keepdims=True)
        xc = x - mu
        sd = jnp.sqrt(jnp.sum(xc * xc, axis=0, keepdims=True) * (1.0 / (_N - 1)))
        xn = xc / (sd + 1e-6)
        xws.append(jnp.dot(xn, wx, preferred_element_type=f32, precision=_PHI))

    g0 = (jnp.dot(ts_ref[...], wiht_ref[...],
                  preferred_element_type=f32, precision=_PHI)
          + lb_ref[...])
    whh = whht_ref[...]

    lstm = {"h": jnp.zeros((1, _H), f32), "c": jnp.zeros((1, _H), f32)}

    def lstm_step(t):
        g = g0[t:t + 1, :] + jnp.dot(lstm["h"], whh,
                                     preferred_element_type=f32, precision=_PHI)
        i = jax.nn.sigmoid(g[:, :_H])
        f = jax.nn.sigmoid(g[:, _H:2 * _H])
        gg = jnp.tanh(g[:, 2 * _H:3 * _H])
        o = jax.nn.sigmoid(g[:, 3 * _H:])
        lstm["c"] = f * lstm["c"] + i * gg
        lstm["h"] = o * jnp.tanh(lstm["c"])

    h = h0_ref[...]
    c = c0_ref[...]
    lsteps_per = _T // _TSEQ
    for t in range(_TSEQ):
        p = xws[t] + jnp.dot(h, wh, preferred_element_type=f32, precision=_PHI)
        z = jnp.dot(adj, p, preferred_element_type=f32, precision=_PHI) + bg
        ig = jax.nn.sigmoid(z[:, :_H])
        fg = jax.nn.sigmoid(z[:, _H:2 * _H])
        og = jax.nn.sigmoid(z[:, 2 * _H:3 * _H])
        mg = jnp.maximum(z[:, 3 * _H:], 0.0)
        c = jnp.tanh(ig * mg + fg * c)
        h = og * jnp.tanh(c)
        for k in range(lsteps_per):
            lstm_step(t * lsteps_per + k)

    pv = pv_ref[...]
    n2 = jnp.sqrt(jnp.sum(pv * pv))
    scol = (jnp.dot(h, pv, preferred_element_type=f32, precision=_PHI)
            * (1.0 / (n2 + 1e-8)))
    mu_s = jnp.mean(scol)
    sd_s = jnp.sqrt(jnp.mean((scol - mu_s) ** 2))
    sig = jax.nn.sigmoid((scol - mu_s) / (sd_s + 1e-8))
    ploss_ref[...] = jnp.mean(sig * (1.0 - sig)).reshape(1, 1)

    srow = lax.dot_general(sig, eye, (((0,), (0,)), ((), ())),
                           preferred_element_type=f32, precision=_PHI)
    gt = (srow > sig).astype(f32)
    eq = (srow == sig).astype(f32)
    lower = (cc < rr).astype(f32)
    rank = jnp.sum(gt + eq * lower, axis=1, keepdims=True)
    selc = (rank < float(_K)).astype(f32)
    selrow = lax.dot_general(selc, eye, (((0,), (0,)), ((), ())),
                             preferred_element_type=f32, precision=_PHI)
    xs = h * sig
    high = jnp.dot(selrow, xs, preferred_element_type=f32, precision=_PHI) * (1.0 / _K)

    fusion = jnp.concatenate([high, lstm["h"]], axis=1)
    mu = jnp.mean(fusion)
    var = jnp.mean((fusion - mu) ** 2)
    fn = (fusion - mu) / jnp.sqrt(var + 1e-5) * lng_ref[...] + lnb_ref[...]
    hmid = jnp.maximum(
        jnp.dot(fn, m1w_ref[...], preferred_element_type=f32, precision=_PHI) + m1b_ref[...],
        0.0)
    pred_ref[...] = (jnp.dot(hmid, m2w_ref[...], preferred_element_type=f32, precision=_PHI)
                     + m2b_ref[...])


_main_call = pl.pallas_call(
    _main_body,
    out_shape=[
        jax.ShapeDtypeStruct((1, 1), jnp.float32),
        jax.ShapeDtypeStruct((1, 1), jnp.float32),
    ],
)


@jax.jit
def kernel(lw_matrixes_sequence, edge_index, hidden_state, cell_state,
           time_series, Wi_x, Wf_x, Wo_x, Wm_x, bi_x, bf_x, bo_x, bm_x,
           Wi_h, Wf_h, Wo_h, Wm_h, bi_h, bf_h, bo_h, bm_h,
           ln_g, ln_b, pool_v, lstm_Wih, lstm_Whh, lstm_bih, lstm_bhh,
           mlp1_W, mlp1_b, mlp2_W, mlp2_b):
    e = edge_index.shape[1]
    eidx = edge_index[1] * _ROWW + edge_index[0]
    npad = _EPAD - e
    j = jnp.arange(npad, dtype=jnp.int32)
    pad_idx = (j % _N) * _ROWW + _N + (j // _N)
    idx_full = jnp.concatenate([eidx, pad_idx]).reshape(_NTILES, _NWIN, _WIN)

    cp = _get_sc_counts()(idx_full).reshape(2, _N, _ROWW)

    wx = jnp.concatenate([Wi_x, Wf_x, Wo_x, Wm_x], axis=1)
    wh = jnp.concatenate([Wi_h, Wf_h, Wo_h, Wm_h], axis=1)
    bgc = jnp.concatenate([bi_x + bi_h, bf_x + bf_h,
                           bo_x + bo_h, bm_x + bm_h]).reshape(1, _G)
    lb = (lstm_bih + lstm_bhh).reshape(1, _G)

    r1 = lambda b: b.reshape(1, b.shape[0])
    pred, ploss = _main_call(
        cp, lw_matrixes_sequence, wx, wh, bgc,
        hidden_state, cell_state, pool_v, time_series,
        lstm_Wih.T, lstm_Whh.T, lb,
        r1(ln_g), r1(ln_b), mlp1_W, r1(mlp1_b), mlp2_W, r1(mlp2_b))

    return pred.reshape(1), ploss[0, 0]
